# baseline (speedup 1.0000x reference)
import functools

import jax
import jax.numpy as jnp
from jax import lax
from jax.experimental import pallas as pl
from jax.experimental.pallas import tpu as pltpu
from jax.experimental.pallas import tpu_sc as plsc

B = 4096
D_IN = 1024
D_LAT = 256
K = 8192
BB = 256


def _dot(a, b, dims=(((1,), (0,)), ((), ())), prec=None):
    return lax.dot_general(a, b, dims, precision=prec,
                           preferred_element_type=jnp.float32)


def _enc_dist_body(a_ref, w1_ref, b1_ref, w2_ref, b2_ref, w3_ref, b3_ref,
                   emb16_ref, emb_ref, z_ref, idx_ref, e2_ref, zp_ref,
                   ii_ref):
    @pl.when(pl.program_id(0) == 0)
    def _():
        e = emb_ref[...]
        e2_ref[...] = _dot(jnp.ones((1, D_LAT), jnp.float32), e * e,
                           dims=(((1,), (1,)), ((), ())),
                           prec=lax.Precision.HIGHEST)
        ii_ref[...] = lax.broadcasted_iota(jnp.int32, (1, K), 1).astype(
            jnp.float32)

    zb = zp_ref[...]
    s = _dot(zb.astype(jnp.bfloat16), emb16_ref[...],
             dims=(((1,), (1,)), ((), ())))
    z2 = jnp.sum(zb * zb, axis=1, keepdims=True)
    d = (z2 - 2.0 * s) + e2_ref[...]
    m = jnp.min(d, axis=1, keepdims=True)
    idx_f = jnp.min(jnp.where(d == m, ii_ref[...], jnp.float32(K)), axis=1)
    idx_ref[...] = idx_f.astype(jnp.int32).reshape(1, 1, BB)

    h = jnp.tanh(_dot(a_ref[...], w1_ref[...]) + b1_ref[...])
    h = jnp.tanh(_dot(h.astype(jnp.bfloat16), w2_ref[...]) + b2_ref[...])
    znew = _dot(h.astype(jnp.bfloat16), w3_ref[...]) + b3_ref[...]
    z_ref[...] = znew
    zp_ref[...] = znew


def _enc_dist(action, w1, b1, w2, b2, w3, b3, emb):
    nb = B // BB
    bf = jnp.bfloat16
    z, idx = pl.pallas_call(
        _enc_dist_body,
        grid=(nb + 1,),
        in_specs=[
            pl.BlockSpec((BB, D_IN), lambda i: (jnp.minimum(i, nb - 1), 0)),
            pl.BlockSpec((D_IN, 2048), lambda i: (0, 0)),
            pl.BlockSpec((1, 2048), lambda i: (0, 0)),
            pl.BlockSpec((2048, 1024), lambda i: (0, 0)),
            pl.BlockSpec((1, 1024), lambda i: (0, 0)),
            pl.BlockSpec((1024, D_LAT), lambda i: (0, 0)),
            pl.BlockSpec((1, D_LAT), lambda i: (0, 0)),
            pl.BlockSpec((K, D_LAT), lambda i: (0, 0)),
            pl.BlockSpec((K, D_LAT), lambda i: (0, 0)),
        ],
        out_specs=[
            pl.BlockSpec((BB, D_LAT), lambda i: (jnp.minimum(i, nb - 1), 0)),
            pl.BlockSpec((1, 1, BB), lambda i: (jnp.maximum(i - 1, 0), 0, 0)),
        ],
        out_shape=[
            jax.ShapeDtypeStruct((B, D_LAT), jnp.float32),
            jax.ShapeDtypeStruct((nb, 1, BB), jnp.int32),
        ],
        scratch_shapes=[pltpu.VMEM((1, K), jnp.float32),
                        pltpu.VMEM((BB, D_LAT), jnp.float32),
                        pltpu.VMEM((1, K), jnp.float32)],
    )(action.astype(bf), w1.astype(bf), b1.reshape(1, -1), w2.astype(bf),
      b2.reshape(1, -1), w3.astype(bf), b3.reshape(1, -1), emb.astype(bf),
      emb)
    return z, idx.reshape(B)


def _make_sc_gather():
    info = plsc.get_sparse_core_info()
    nc, ns = info.num_cores, info.num_subcores
    bw = B // (nc * ns)

    @functools.partial(
        pl.kernel,
        mesh=plsc.VectorSubcoreMesh(core_axis_name="c", subcore_axis_name="s"),
        out_type=jax.ShapeDtypeStruct((B, D_LAT), jnp.float32),
        scratch_types=[
            pltpu.VMEM((bw,), jnp.int32),
            pltpu.VMEM((bw, D_LAT), jnp.float32),
            pltpu.SemaphoreType.DMA,
        ],
    )
    def gather_k(emb_hbm, idx_hbm, out_hbm, idx_v, rows_v, sem):
        wid = lax.axis_index("s") * nc + lax.axis_index("c")
        base = wid * bw
        pltpu.sync_copy(idx_hbm.at[pl.ds(base, bw)], idx_v)
        pltpu.async_copy(emb_hbm.at[idx_v], rows_v, sem).wait()
        pltpu.sync_copy(rows_v, out_hbm.at[pl.ds(base, bw)])

    return gather_k


def _dec_body(z_ref, q_ref, w1_ref, b1_ref, w2_ref, b2_ref, w3_ref, b3_ref,
              recon_ref, acc_ref, w2s_ref, w3s_ref):
    zb = z_ref[...]
    qb = q_ref[...]

    @pl.when(pl.program_id(0) == 0)
    def _():
        acc_ref[...] = jnp.zeros_like(acc_ref)
        w2s_ref[...] = w2_ref[...].astype(jnp.bfloat16)
        w3s_ref[...] = w3_ref[...].astype(jnp.bfloat16)

    dz = zb - qb
    acc_ref[...] += jnp.sum(dz * dz).reshape(1, 1)

    h = jnp.tanh(_dot(qb, w1_ref[...]) + b1_ref[...])
    h = jnp.tanh(_dot(h.astype(jnp.bfloat16), w2s_ref[...]) + b2_ref[...])
    recon_ref[...] = _dot(h.astype(jnp.bfloat16), w3s_ref[...]) + b3_ref[...]


def _decode(z, q, w1, b1, w2, b2, w3, b3):
    recon, acc = pl.pallas_call(
        _dec_body,
        grid=(B // BB,),
        in_specs=[
            pl.BlockSpec((BB, D_LAT), lambda i: (i, 0)),
            pl.BlockSpec((BB, D_LAT), lambda i: (i, 0)),
            pl.BlockSpec((D_LAT, 1024), lambda i: (0, 0)),
            pl.BlockSpec((1, 1024), lambda i: (0, 0)),
            pl.BlockSpec((1024, 2048), lambda i: (0, 0)),
            pl.BlockSpec((1, 2048), lambda i: (0, 0)),
            pl.BlockSpec((2048, D_IN), lambda i: (0, 0)),
            pl.BlockSpec((1, D_IN), lambda i: (0, 0)),
        ],
        out_specs=[
            pl.BlockSpec((BB, D_IN), lambda i: (i, 0)),
            pl.BlockSpec((1, 1), lambda i: (0, 0)),
        ],
        out_shape=[
            jax.ShapeDtypeStruct((B, D_IN), jnp.float32),
            jax.ShapeDtypeStruct((1, 1), jnp.float32),
        ],
        scratch_shapes=[pltpu.VMEM((1024, 2048), jnp.bfloat16),
                        pltpu.VMEM((2048, D_IN), jnp.bfloat16)],
    )(z, q, w1, b1.reshape(1, -1), w2, b2.reshape(1, -1), w3,
      b3.reshape(1, -1))
    return recon, acc


def kernel(state, action, enc_w1, enc_b1, enc_w2, enc_b2, enc_w3, enc_b3, emb,
           dec_w1, dec_b1, dec_w2, dec_b2, dec_w3, dec_b3):
    z, idx = _enc_dist(action, enc_w1, enc_b1, enc_w2, enc_b2, enc_w3, enc_b3,
                       emb)
    q_st = _make_sc_gather()(emb, idx)
    recon, acc = _decode(z, q_st, dec_w1, dec_b1, dec_w2, dec_b2, dec_w3,
                         dec_b3)
    mean_sq = acc[0, 0] / (B * D_LAT)
    vq_loss = 0.25 * mean_sq + mean_sq
    return recon, z, q_st, vq_loss


# speedup vs baseline: 1.1224x; 1.1224x over previous
import functools

import jax
import jax.numpy as jnp
from jax import lax
from jax.experimental import pallas as pl
from jax.experimental.pallas import tpu as pltpu
from jax.experimental.pallas import tpu_sc as plsc

B = 4096
D_IN = 1024
D_LAT = 256
K = 8192
BB = 256


def _dot(a, b, dims=(((1,), (0,)), ((), ())), prec=None):
    return lax.dot_general(a, b, dims, precision=prec,
                           preferred_element_type=jnp.float32)


def _enc_dist_body(a_ref, w1_ref, b1_ref, w2_ref, b2_ref, w3_ref, b3_ref,
                   emb_ref, z_ref, idx_ref, e2_ref, ii_ref):
    @pl.when(pl.program_id(0) == 0)
    def _():
        e = emb_ref[...]
        e2_ref[...] = _dot(jnp.ones((1, D_LAT), jnp.float32), e * e,
                           dims=(((1,), (1,)), ((), ())),
                           prec=lax.Precision.HIGHEST)
        ii_ref[...] = lax.broadcasted_iota(jnp.int32, (1, K), 1).astype(
            jnp.float32)

    h = jnp.tanh(_dot(a_ref[...], w1_ref[...]) + b1_ref[...])
    h = jnp.tanh(_dot(h, w2_ref[...]) + b2_ref[...])
    zb = _dot(h, w3_ref[...]) + b3_ref[...]
    z_ref[...] = zb

    s = _dot(zb, emb_ref[...], dims=(((1,), (1,)), ((), ())))
    z2 = jnp.sum(zb * zb, axis=1, keepdims=True)
    d = (z2 - 2.0 * s) + e2_ref[...]
    m = jnp.min(d, axis=1, keepdims=True)
    idx_f = jnp.min(jnp.where(d == m, ii_ref[...], jnp.float32(K)), axis=1)
    idx_ref[...] = idx_f.astype(jnp.int32).reshape(1, 1, BB)


def _enc_dist(action, w1, b1, w2, b2, w3, b3, emb):
    z, idx = pl.pallas_call(
        _enc_dist_body,
        grid=(B // BB,),
        in_specs=[
            pl.BlockSpec((BB, D_IN), lambda i: (i, 0)),
            pl.BlockSpec((D_IN, 2048), lambda i: (0, 0)),
            pl.BlockSpec((1, 2048), lambda i: (0, 0)),
            pl.BlockSpec((2048, 1024), lambda i: (0, 0)),
            pl.BlockSpec((1, 1024), lambda i: (0, 0)),
            pl.BlockSpec((1024, D_LAT), lambda i: (0, 0)),
            pl.BlockSpec((1, D_LAT), lambda i: (0, 0)),
            pl.BlockSpec((K, D_LAT), lambda i: (0, 0)),
        ],
        out_specs=[
            pl.BlockSpec((BB, D_LAT), lambda i: (i, 0)),
            pl.BlockSpec((1, 1, BB), lambda i: (i, 0, 0)),
        ],
        out_shape=[
            jax.ShapeDtypeStruct((B, D_LAT), jnp.float32),
            jax.ShapeDtypeStruct((B // BB, 1, BB), jnp.int32),
        ],
        scratch_shapes=[pltpu.VMEM((1, K), jnp.float32),
                        pltpu.VMEM((1, K), jnp.float32)],
    )(action, w1, b1.reshape(1, -1), w2, b2.reshape(1, -1), w3,
      b3.reshape(1, -1), emb)
    return z, idx.reshape(B)


def _make_sc_gather():
    info = plsc.get_sparse_core_info()
    nc, ns = info.num_cores, info.num_subcores
    bw = B // (nc * ns)

    @functools.partial(
        pl.kernel,
        mesh=plsc.VectorSubcoreMesh(core_axis_name="c", subcore_axis_name="s"),
        out_type=jax.ShapeDtypeStruct((B, D_LAT), jnp.float32),
        scratch_types=[
            pltpu.VMEM((bw,), jnp.int32),
            pltpu.VMEM((bw, D_LAT), jnp.float32),
            pltpu.SemaphoreType.DMA,
        ],
    )
    def gather_k(emb_hbm, idx_hbm, out_hbm, idx_v, rows_v, sem):
        wid = lax.axis_index("s") * nc + lax.axis_index("c")
        base = wid * bw
        pltpu.sync_copy(idx_hbm.at[pl.ds(base, bw)], idx_v)
        pltpu.async_copy(emb_hbm.at[idx_v], rows_v, sem).wait()
        pltpu.sync_copy(rows_v, out_hbm.at[pl.ds(base, bw)])

    return gather_k


def _dec_body(z_ref, q_ref, w1_ref, b1_ref, w2_ref, b2_ref, w3_ref, b3_ref,
              recon_ref, acc_ref):
    zb = z_ref[...]
    qb = q_ref[...]

    @pl.when(pl.program_id(0) == 0)
    def _():
        acc_ref[...] = jnp.zeros_like(acc_ref)

    dz = zb - qb
    acc_ref[...] += jnp.sum(dz * dz).reshape(1, 1)

    h = jnp.tanh(_dot(qb, w1_ref[...]) + b1_ref[...])
    h = jnp.tanh(_dot(h, w2_ref[...]) + b2_ref[...])
    recon_ref[...] = _dot(h, w3_ref[...]) + b3_ref[...]


def _decode(z, q, w1, b1, w2, b2, w3, b3):
    recon, acc = pl.pallas_call(
        _dec_body,
        grid=(B // BB,),
        in_specs=[
            pl.BlockSpec((BB, D_LAT), lambda i: (i, 0)),
            pl.BlockSpec((BB, D_LAT), lambda i: (i, 0)),
            pl.BlockSpec((D_LAT, 1024), lambda i: (0, 0)),
            pl.BlockSpec((1, 1024), lambda i: (0, 0)),
            pl.BlockSpec((1024, 2048), lambda i: (0, 0)),
            pl.BlockSpec((1, 2048), lambda i: (0, 0)),
            pl.BlockSpec((2048, D_IN), lambda i: (0, 0)),
            pl.BlockSpec((1, D_IN), lambda i: (0, 0)),
        ],
        out_specs=[
            pl.BlockSpec((BB, D_IN), lambda i: (i, 0)),
            pl.BlockSpec((1, 1), lambda i: (0, 0)),
        ],
        out_shape=[
            jax.ShapeDtypeStruct((B, D_IN), jnp.float32),
            jax.ShapeDtypeStruct((1, 1), jnp.float32),
        ],
    )(z, q, w1, b1.reshape(1, -1), w2, b2.reshape(1, -1), w3,
      b3.reshape(1, -1))
    return recon, acc


def kernel(state, action, enc_w1, enc_b1, enc_w2, enc_b2, enc_w3, enc_b3, emb,
           dec_w1, dec_b1, dec_w2, dec_b2, dec_w3, dec_b3):
    z, idx = _enc_dist(action, enc_w1, enc_b1, enc_w2, enc_b2, enc_w3, enc_b3,
                       emb)
    q_st = _make_sc_gather()(emb, idx)
    recon, acc = _decode(z, q_st, dec_w1, dec_b1, dec_w2, dec_b2, dec_w3,
                         dec_b3)
    mean_sq = acc[0, 0] / (B * D_LAT)
    vq_loss = 0.25 * mean_sq + mean_sq
    return recon, z, q_st, vq_loss
